# Initial kernel scaffold; baseline (speedup 1.0000x reference)
import jax
import jax.numpy as jnp
from jax.experimental import pallas as pl


def kernel(x, edge_index, W1, b1, W2, b2):
    raise NotImplementedError("write your pallas kernel here")


# speedup vs baseline: 15.7106x; 15.7106x over previous
import functools

import jax
import jax.numpy as jnp
from jax import lax
from jax.experimental import pallas as pl
from jax.experimental.pallas import tpu as pltpu
from jax.experimental.pallas import tpu_sc as plsc

N_NODES = 10000
N_EDGES = 320000
F_IN = 128
F_HID = 128
F_LAT = 64

NC = 2
NS = 16
NW = NC * NS

NP = 10240
RPT = NP // NS
CHUNK = 128
K_CHUNKS = 79
EW = CHUNK * K_CHUNKS
EP = NW * EW

_MESH = plsc.VectorSubcoreMesh(
    core_axis_name="c", subcore_axis_name="s", num_cores=NC, num_subcores=NS)


def _hist_body(dst_hbm, ones_hbm, zeros_hbm, out_hbm, acc, onev, idxv):
    cid = lax.axis_index("c")
    sid = lax.axis_index("s")
    wid = cid * NS + sid
    base = wid * EW
    pltpu.sync_copy(zeros_hbm, acc.at[pl.ds(sid * RPT, RPT)])
    pltpu.sync_copy(ones_hbm, onev)
    plsc.subcore_barrier()

    def step(k, carry):
        pltpu.sync_copy(dst_hbm.at[pl.ds(base + k * CHUNK, CHUNK)], idxv)
        pltpu.sync_copy(onev, acc.at[idxv], add=True)
        return carry

    lax.fori_loop(0, K_CHUNKS, step, 0)
    plsc.subcore_barrier()
    pltpu.sync_copy(acc.at[pl.ds(sid * RPT, RPT)],
                    out_hbm.at[cid, pl.ds(sid * RPT, RPT)])


_hist = pl.kernel(
    _hist_body,
    out_type=jax.ShapeDtypeStruct((NC, NP, 1), jnp.float32),
    mesh=_MESH,
    scratch_types=[
        pltpu.VMEM_SHARED((NP, 1), jnp.float32),
        pltpu.VMEM((CHUNK, 1), jnp.float32),
        pltpu.VMEM((CHUNK,), jnp.int32),
    ],
)


def _spmm_body(f, t_hbm, src_hbm, dst_hbm, zeros_hbm, out_hbm,
               acc, srcv, dstv, rows, sem):
    cid = lax.axis_index("c")
    sid = lax.axis_index("s")
    wid = cid * NS + sid
    base = wid * EW
    pltpu.sync_copy(zeros_hbm, acc.at[pl.ds(sid * RPT, RPT)])
    plsc.subcore_barrier()

    def step(k, carry):
        off = base + k * CHUNK
        pltpu.sync_copy(src_hbm.at[pl.ds(off, CHUNK)], srcv)
        pltpu.sync_copy(dst_hbm.at[pl.ds(off, CHUNK)], dstv)
        pltpu.async_copy(t_hbm.at[srcv], rows, sem).wait()
        pltpu.sync_copy(rows, acc.at[dstv], add=True)
        return carry

    lax.fori_loop(0, K_CHUNKS, step, 0)
    plsc.subcore_barrier()
    pltpu.sync_copy(acc.at[pl.ds(sid * RPT, RPT)],
                    out_hbm.at[cid, pl.ds(sid * RPT, RPT)])


def _make_spmm(f):
    return pl.kernel(
        functools.partial(_spmm_body, f),
        out_type=jax.ShapeDtypeStruct((NC, NP, f), jnp.float32),
        mesh=_MESH,
        scratch_types=[
            pltpu.VMEM_SHARED((NP, f), jnp.float32),
            pltpu.VMEM((CHUNK,), jnp.int32),
            pltpu.VMEM((CHUNK,), jnp.int32),
            pltpu.VMEM((CHUNK, f), jnp.float32),
            pltpu.SemaphoreType.DMA,
        ],
    )


_spmm_128 = _make_spmm(F_HID)


def _dinv_of(degp_ref):
    deg = degp_ref[0] + degp_ref[1] + 1.0
    return lax.rsqrt(deg)


def _tc_a_body(x_ref, w_ref, degp_ref, t_ref):
    dinv = _dinv_of(degp_ref)
    h = jnp.dot(x_ref[...], w_ref[...], preferred_element_type=jnp.float32)
    t_ref[...] = h * dinv


def _tc_b_body(sp_ref, t1_ref, degp_ref, b1_ref, t2_ref):
    dinv = _dinv_of(degp_ref)
    s = sp_ref[0] + sp_ref[1] + t1_ref[...]
    h = jnp.maximum(s * dinv + b1_ref[...], 0.0)
    t2_ref[...] = h * dinv


def _tc_c_body(sp_ref, t2_ref, degp_ref, b2_ref, w2_ref, o_ref):
    dinv = _dinv_of(degp_ref)
    s = (sp_ref[0] + sp_ref[1] + t2_ref[...]) * dinv
    o_ref[...] = jnp.dot(s, w2_ref[...],
                         preferred_element_type=jnp.float32) + b2_ref[...]


_tc_a = pl.pallas_call(
    _tc_a_body, out_shape=jax.ShapeDtypeStruct((NP, F_HID), jnp.float32))
_tc_b = pl.pallas_call(
    _tc_b_body, out_shape=jax.ShapeDtypeStruct((NP, F_HID), jnp.float32))
_tc_c = pl.pallas_call(
    _tc_c_body, out_shape=jax.ShapeDtypeStruct((NP, F_LAT), jnp.float32))


def kernel(x, edge_index, W1, b1, W2, b2):
    src = edge_index[0]
    dst = edge_index[1]
    n_pad = EP - N_EDGES
    pad_ids = (jnp.arange(n_pad, dtype=jnp.int32) % (NP - N_NODES)) + N_NODES
    srcp = jnp.concatenate([src, pad_ids])
    dstp = jnp.concatenate([dst, pad_ids])
    x_pad = jnp.zeros((NP, F_IN), jnp.float32).at[:N_NODES].set(x)

    ones_col = jnp.ones((CHUNK, 1), jnp.float32)
    zeros_col = jnp.zeros((RPT, 1), jnp.float32)
    zeros_128 = jnp.zeros((RPT, F_HID), jnp.float32)

    degp = _hist(dstp, ones_col, zeros_col)
    t1 = _tc_a(x_pad, W1, degp)
    s1p = _spmm_128(t1, srcp, dstp, zeros_128)
    t2 = _tc_b(s1p, t1, degp, b1.reshape(1, F_HID))
    s2p = _spmm_128(t2, srcp, dstp, zeros_128)
    out = _tc_c(s2p, t2, degp, b2.reshape(1, F_LAT), W2)
    return out[:N_NODES]
